# baseline (speedup 1.0000x reference)
import functools

import jax
import jax.numpy as jnp
from jax import lax
from jax.experimental import pallas as pl
from jax.experimental.pallas import tpu as pltpu
from jax.experimental.pallas import tpu_sc as plsc

BATCH = 4096
MAXLEN = 200
EMBED = 64
N_TOK = BATCH * MAXLEN

NC, NS = 2, 16
NW = NC * NS
TOK_PER_W = N_TOK // NW

SEQ_PER_CHUNK = 4
CHUNK = SEQ_PER_CHUNK * MAXLEN
N_CHUNK = TOK_PER_W // CHUNK
IDX_MINOR = 100
IDX_ROWS = CHUNK // IDX_MINOR
LANES = 16
VPE = EMBED // LANES


def _body(x_hbm, tab_hbm, pos_hbm, out_hbm, idx_v, rows_v, pos_v, sem):
    wid = lax.axis_index("s") * NC + lax.axis_index("c")
    base = wid * TOK_PER_W

    pltpu.sync_copy(pos_hbm, pos_v)

    def chunk_body(c, carry):
        tok0 = base + c * CHUNK
        row0 = pl.multiple_of(tok0 // IDX_MINOR, IDX_ROWS)
        pltpu.sync_copy(x_hbm.at[pl.ds(row0, IDX_ROWS)], idx_v)
        copies = [
            pltpu.make_async_copy(
                tab_hbm.at[idx_v.at[j]],
                rows_v.at[pl.ds(j * IDX_MINOR, IDX_MINOR)],
                sem,
            )
            for j in range(IDX_ROWS)
        ]
        for cp in copies:
            cp.start()
        for cp in copies:
            cp.wait()

        def pos_body(p, carry2):
            pv = [pos_v[p, pl.ds(j * LANES, LANES)] for j in range(VPE)]
            for s in range(SEQ_PER_CHUNK):
                r = s * MAXLEN + p
                for j in range(VPE):
                    rows_v[r, pl.ds(j * LANES, LANES)] += pv[j]
            return carry2

        lax.fori_loop(0, MAXLEN, pos_body, 0)

        pltpu.sync_copy(rows_v, out_hbm.at[pl.ds(tok0, CHUNK)])
        return carry

    lax.fori_loop(0, N_CHUNK, chunk_body, 0)


@functools.partial(jax.jit, static_argnames=())
def _run(x2d, token_table, pos_table):
    mesh = plsc.VectorSubcoreMesh(core_axis_name="c", subcore_axis_name="s")
    f = functools.partial(
        pl.kernel,
        out_type=jax.ShapeDtypeStruct((N_TOK, EMBED), jnp.float32),
        mesh=mesh,
        scratch_types=[
            pltpu.VMEM((IDX_ROWS, IDX_MINOR), jnp.int32),
            pltpu.VMEM((CHUNK, EMBED), jnp.float32),
            pltpu.VMEM((MAXLEN, EMBED), jnp.float32),
            pltpu.SemaphoreType.DMA,
        ],
        compiler_params=pltpu.CompilerParams(use_tc_tiling_on_sc=False),
    )(_body)
    return f(x2d, token_table, pos_table)


def kernel(x, token_table, pos_table):
    x2d = x.astype(jnp.int32).reshape(N_TOK // IDX_MINOR, IDX_MINOR)
    out = _run(x2d, token_table, pos_table)
    return out.reshape(BATCH, MAXLEN, EMBED)


# speedup vs baseline: 1.0666x; 1.0666x over previous
import functools

import jax
import jax.numpy as jnp
from jax import lax
from jax.experimental import pallas as pl
from jax.experimental.pallas import tpu as pltpu
from jax.experimental.pallas import tpu_sc as plsc

BATCH = 4096
MAXLEN = 200
EMBED = 64
N_TOK = BATCH * MAXLEN

NC, NS = 2, 16
NW = NC * NS
TOK_PER_W = N_TOK // NW

SEQ_PER_CHUNK = 4
CHUNK = SEQ_PER_CHUNK * MAXLEN
N_CHUNK = TOK_PER_W // CHUNK
IDX_MINOR = 100
IDX_ROWS = CHUNK // IDX_MINOR
LANES = 16
VPE = EMBED // LANES
NBUF = 2


def _start_chunk(x_hbm, tab_hbm, idx_v, rows_v, gsem, tok0):
    row0 = pl.multiple_of(tok0 // IDX_MINOR, IDX_ROWS)
    pltpu.sync_copy(x_hbm.at[pl.ds(row0, IDX_ROWS)], idx_v)
    for j in range(IDX_ROWS):
        pltpu.make_async_copy(
            tab_hbm.at[idx_v.at[j]],
            rows_v.at[pl.ds(j * IDX_MINOR, IDX_MINOR)],
            gsem,
        ).start()


def _wait_chunk(tab_hbm, idx_v, rows_v, gsem):
    for j in range(IDX_ROWS):
        pltpu.make_async_copy(
            tab_hbm.at[idx_v.at[j]],
            rows_v.at[pl.ds(j * IDX_MINOR, IDX_MINOR)],
            gsem,
        ).wait()


def _add_pos(rows_v, pos_v):
    @pl.loop(0, MAXLEN)
    def pos_body(p):
        pv = [pos_v[p, pl.ds(j * LANES, LANES)] for j in range(VPE)]
        for s in range(SEQ_PER_CHUNK):
            r = s * MAXLEN + p
            for j in range(VPE):
                rows_v[r, pl.ds(j * LANES, LANES)] += pv[j]


def _body(x_hbm, tab_hbm, pos_hbm, out_hbm,
          idx0, idx1, rows0, rows1, pos_v, gsem0, gsem1, ssem0, ssem1):
    wid = lax.axis_index("s") * NC + lax.axis_index("c")
    base = wid * TOK_PER_W
    idx = [idx0, idx1]
    rows = [rows0, rows1]
    gsem = [gsem0, gsem1]
    ssem = [ssem0, ssem1]

    pltpu.sync_copy(pos_hbm, pos_v)

    for b in range(NBUF):
        _start_chunk(x_hbm, tab_hbm, idx[b], rows[b], gsem[b], base + b * CHUNK)

    @pl.loop(0, N_CHUNK, step=NBUF)
    def chunk_body(c):
        for b in range(NBUF):
            tok0 = base + (c + b) * CHUNK
            _wait_chunk(tab_hbm, idx[b], rows[b], gsem[b])
            _add_pos(rows[b], pos_v)
            st = pltpu.make_async_copy(
                rows[b], out_hbm.at[pl.ds(tok0, CHUNK)], ssem[b])
            st.start()
            @pl.when(c + b + NBUF < N_CHUNK)
            def _():
                st.wait()
                _start_chunk(x_hbm, tab_hbm, idx[b], rows[b], gsem[b],
                             tok0 + NBUF * CHUNK)

    for b in range(NBUF):
        tok_last = base + (N_CHUNK - NBUF + b) * CHUNK
        pltpu.make_async_copy(
            rows[b], out_hbm.at[pl.ds(tok_last, CHUNK)], ssem[b]).wait()


@jax.jit
def _run(x2d, token_table, pos_table):
    mesh = plsc.VectorSubcoreMesh(core_axis_name="c", subcore_axis_name="s")
    f = functools.partial(
        pl.kernel,
        out_type=jax.ShapeDtypeStruct((N_TOK, EMBED), jnp.float32),
        mesh=mesh,
        scratch_types=[
            pltpu.VMEM((IDX_ROWS, IDX_MINOR), jnp.int32),
            pltpu.VMEM((IDX_ROWS, IDX_MINOR), jnp.int32),
            pltpu.VMEM((CHUNK, EMBED), jnp.float32),
            pltpu.VMEM((CHUNK, EMBED), jnp.float32),
            pltpu.VMEM((MAXLEN, EMBED), jnp.float32),
            pltpu.SemaphoreType.DMA,
            pltpu.SemaphoreType.DMA,
            pltpu.SemaphoreType.DMA,
            pltpu.SemaphoreType.DMA,
        ],
        compiler_params=pltpu.CompilerParams(use_tc_tiling_on_sc=False),
    )(_body)
    return f(x2d, token_table, pos_table)


def kernel(x, token_table, pos_table):
    x2d = x.astype(jnp.int32).reshape(N_TOK // IDX_MINOR, IDX_MINOR)
    out = _run(x2d, token_table, pos_table)
    return out.reshape(BATCH, MAXLEN, EMBED)


# speedup vs baseline: 1.0670x; 1.0004x over previous
import functools

import jax
import jax.numpy as jnp
from jax import lax
from jax.experimental import pallas as pl
from jax.experimental.pallas import tpu as pltpu
from jax.experimental.pallas import tpu_sc as plsc

BATCH = 4096
MAXLEN = 200
EMBED = 64

NC, NS = 2, 16
NW = NC * NS
SEQ_PER_W = BATCH // NW

SEQ_PER_CHUNK = 4
N_CHUNK = SEQ_PER_W // SEQ_PER_CHUNK
CHUNK = SEQ_PER_CHUNK * MAXLEN
SPLITS = ((0, 128), (128, 72))
LANES = 16
VPE = EMBED // LANES
NBUF = 2


def _start_chunk(x_hbm, tab_hbm, idx_v, rows_v, gsem, seq0):
    pltpu.sync_copy(x_hbm.at[pl.ds(seq0, SEQ_PER_CHUNK)], idx_v)
    for s in range(SEQ_PER_CHUNK):
        for off, ln in SPLITS:
            pltpu.make_async_copy(
                tab_hbm.at[idx_v.at[s, pl.ds(off, ln)]],
                rows_v.at[pl.ds(s * MAXLEN + off, ln)],
                gsem,
            ).start()


def _wait_chunk(tab_hbm, idx_v, rows_v, gsem):
    for s in range(SEQ_PER_CHUNK):
        for off, ln in SPLITS:
            pltpu.make_async_copy(
                tab_hbm.at[idx_v.at[s, pl.ds(off, ln)]],
                rows_v.at[pl.ds(s * MAXLEN + off, ln)],
                gsem,
            ).wait()


def _add_pos(rows_v, pos_v):
    @pl.loop(0, MAXLEN)
    def pos_body(p):
        pv = [pos_v[p, pl.ds(j * LANES, LANES)] for j in range(VPE)]
        for s in range(SEQ_PER_CHUNK):
            r = s * MAXLEN + p
            for j in range(VPE):
                rows_v[r, pl.ds(j * LANES, LANES)] += pv[j]


def _store_chunk(rows_v, out_hbm, ssem, seq0, start):
    for s in range(SEQ_PER_CHUNK):
        cp = pltpu.make_async_copy(
            rows_v.at[pl.ds(s * MAXLEN, MAXLEN)],
            out_hbm.at[seq0 + s],
            ssem,
        )
        if start:
            cp.start()
        else:
            cp.wait()


def _body(x_hbm, tab_hbm, pos_hbm, out_hbm,
          idx0, idx1, rows0, rows1, pos_v, gsem0, gsem1, ssem0, ssem1):
    wid = lax.axis_index("s") * NC + lax.axis_index("c")
    base = wid * SEQ_PER_W
    idx = [idx0, idx1]
    rows = [rows0, rows1]
    gsem = [gsem0, gsem1]
    ssem = [ssem0, ssem1]

    pltpu.sync_copy(pos_hbm, pos_v)

    for b in range(NBUF):
        _start_chunk(x_hbm, tab_hbm, idx[b], rows[b], gsem[b],
                     base + b * SEQ_PER_CHUNK)

    @pl.loop(0, N_CHUNK, step=NBUF)
    def chunk_body(c):
        for b in range(NBUF):
            seq0 = base + (c + b) * SEQ_PER_CHUNK
            _wait_chunk(tab_hbm, idx[b], rows[b], gsem[b])
            _add_pos(rows[b], pos_v)
            _store_chunk(rows[b], out_hbm, ssem[b], seq0, start=True)
            @pl.when(c + b + NBUF < N_CHUNK)
            def _():
                _store_chunk(rows[b], out_hbm, ssem[b], seq0, start=False)
                _start_chunk(x_hbm, tab_hbm, idx[b], rows[b], gsem[b],
                             seq0 + NBUF * SEQ_PER_CHUNK)

    for b in range(NBUF):
        seq_last = base + (N_CHUNK - NBUF + b) * SEQ_PER_CHUNK
        _store_chunk(rows[b], out_hbm, ssem[b], seq_last, start=False)


@jax.jit
def _run(x, token_table, pos_table):
    mesh = plsc.VectorSubcoreMesh(core_axis_name="c", subcore_axis_name="s")
    f = functools.partial(
        pl.kernel,
        out_type=jax.ShapeDtypeStruct((BATCH, MAXLEN, EMBED), jnp.float32),
        mesh=mesh,
        scratch_types=[
            pltpu.VMEM((SEQ_PER_CHUNK, MAXLEN), jnp.int32),
            pltpu.VMEM((SEQ_PER_CHUNK, MAXLEN), jnp.int32),
            pltpu.VMEM((CHUNK, EMBED), jnp.float32),
            pltpu.VMEM((CHUNK, EMBED), jnp.float32),
            pltpu.VMEM((MAXLEN, EMBED), jnp.float32),
            pltpu.SemaphoreType.DMA,
            pltpu.SemaphoreType.DMA,
            pltpu.SemaphoreType.DMA,
            pltpu.SemaphoreType.DMA,
        ],
        compiler_params=pltpu.CompilerParams(use_tc_tiling_on_sc=False),
    )(_body)
    return f(x, token_table, pos_table)


def kernel(x, token_table, pos_table):
    return _run(x.astype(jnp.int32), token_table, pos_table)


# speedup vs baseline: 1.0770x; 1.0093x over previous
import functools

import jax
import jax.numpy as jnp
from jax import lax
from jax.experimental import pallas as pl
from jax.experimental.pallas import tpu as pltpu
from jax.experimental.pallas import tpu_sc as plsc

BATCH = 4096
MAXLEN = 200
EMBED = 64

NC, NS = 2, 16
NW = NC * NS
SEQ_PER_W = BATCH // NW

SEQ_PER_CHUNK = 2
N_CHUNK = SEQ_PER_W // SEQ_PER_CHUNK
CHUNK = SEQ_PER_CHUNK * MAXLEN
SPLITS = ((0, 128), (128, 72))
LANES = 16
VPE = EMBED // LANES
NBUF = 4
LOOKAHEAD = 2


def _start_chunk(x_hbm, tab_hbm, idx_v, rows_v, gsem, seq0):
    pltpu.sync_copy(x_hbm.at[pl.ds(seq0, SEQ_PER_CHUNK)], idx_v)
    for s in range(SEQ_PER_CHUNK):
        for off, ln in SPLITS:
            pltpu.make_async_copy(
                tab_hbm.at[idx_v.at[s, pl.ds(off, ln)]],
                rows_v.at[pl.ds(s * MAXLEN + off, ln)],
                gsem,
            ).start()


def _wait_chunk(tab_hbm, idx_v, rows_v, gsem):
    for s in range(SEQ_PER_CHUNK):
        for off, ln in SPLITS:
            pltpu.make_async_copy(
                tab_hbm.at[idx_v.at[s, pl.ds(off, ln)]],
                rows_v.at[pl.ds(s * MAXLEN + off, ln)],
                gsem,
            ).wait()


def _add_pos(rows_v, pos_v):
    @pl.loop(0, MAXLEN)
    def pos_body(p):
        pv = [pos_v[p, pl.ds(j * LANES, LANES)] for j in range(VPE)]
        for s in range(SEQ_PER_CHUNK):
            r = s * MAXLEN + p
            for j in range(VPE):
                rows_v[r, pl.ds(j * LANES, LANES)] += pv[j]


def _store_chunk(rows_v, out_hbm, ssem, seq0, start):
    for s in range(SEQ_PER_CHUNK):
        cp = pltpu.make_async_copy(
            rows_v.at[pl.ds(s * MAXLEN, MAXLEN)],
            out_hbm.at[seq0 + s],
            ssem,
        )
        if start:
            cp.start()
        else:
            cp.wait()


def _body(x_hbm, tab_hbm, pos_hbm, out_hbm,
          idx0, idx1, idx2, idx3, rows0, rows1, rows2, rows3, pos_v,
          gsem0, gsem1, gsem2, gsem3, ssem0, ssem1, ssem2, ssem3):
    wid = lax.axis_index("s") * NC + lax.axis_index("c")
    base = wid * SEQ_PER_W
    idx = [idx0, idx1, idx2, idx3]
    rows = [rows0, rows1, rows2, rows3]
    gsem = [gsem0, gsem1, gsem2, gsem3]
    ssem = [ssem0, ssem1, ssem2, ssem3]

    pltpu.sync_copy(pos_hbm, pos_v)

    for b in range(LOOKAHEAD):
        _start_chunk(x_hbm, tab_hbm, idx[b], rows[b], gsem[b],
                     base + b * SEQ_PER_CHUNK)

    @pl.loop(0, N_CHUNK, step=NBUF)
    def chunk_body(c):
        for b in range(NBUF):
            seq0 = base + (c + b) * SEQ_PER_CHUNK
            _wait_chunk(tab_hbm, idx[b], rows[b], gsem[b])
            _add_pos(rows[b], pos_v)
            _store_chunk(rows[b], out_hbm, ssem[b], seq0, start=True)
            b2 = (b + LOOKAHEAD) % NBUF

            @pl.when(c + b + LOOKAHEAD < N_CHUNK)
            def _():
                @pl.when(c + b + LOOKAHEAD >= NBUF)
                def _():
                    _store_chunk(
                        rows[b2], out_hbm, ssem[b2],
                        base + (c + b + LOOKAHEAD - NBUF) * SEQ_PER_CHUNK,
                        start=False)
                _start_chunk(x_hbm, tab_hbm, idx[b2], rows[b2], gsem[b2],
                             base + (c + b + LOOKAHEAD) * SEQ_PER_CHUNK)

    for b in range(NBUF):
        cc = N_CHUNK - NBUF + b
        _store_chunk(rows[cc % NBUF], out_hbm, ssem[cc % NBUF],
                     base + cc * SEQ_PER_CHUNK, start=False)


@jax.jit
def _run(x, token_table, pos_table):
    mesh = plsc.VectorSubcoreMesh(core_axis_name="c", subcore_axis_name="s")
    f = functools.partial(
        pl.kernel,
        out_type=jax.ShapeDtypeStruct((BATCH, MAXLEN, EMBED), jnp.float32),
        mesh=mesh,
        scratch_types=(
            [pltpu.VMEM((SEQ_PER_CHUNK, MAXLEN), jnp.int32)] * NBUF
            + [pltpu.VMEM((CHUNK, EMBED), jnp.float32)] * NBUF
            + [pltpu.VMEM((MAXLEN, EMBED), jnp.float32)]
            + [pltpu.SemaphoreType.DMA] * (2 * NBUF)
        ),
        compiler_params=pltpu.CompilerParams(use_tc_tiling_on_sc=False),
    )(_body)
    return f(x, token_table, pos_table)


def kernel(x, token_table, pos_table):
    return _run(x.astype(jnp.int32), token_table, pos_table)


# speedup vs baseline: 1.0785x; 1.0014x over previous
import functools

import jax
import jax.numpy as jnp
from jax import lax
from jax.experimental import pallas as pl
from jax.experimental.pallas import tpu as pltpu
from jax.experimental.pallas import tpu_sc as plsc

BATCH = 4096
MAXLEN = 200
EMBED = 64

NC, NS = 2, 16
NW = NC * NS
SEQ_PER_W = BATCH // NW

SEQ_PER_CHUNK = 2
N_CHUNK = SEQ_PER_W // SEQ_PER_CHUNK
CHUNK = SEQ_PER_CHUNK * MAXLEN
SPLITS = ((0, 128), (128, 72))
LANES = 16
VPE = EMBED // LANES
NBUF = 4
LOOKAHEAD = 3


def _start_chunk(x_hbm, tab_hbm, idx_v, rows_v, gsem, seq0):
    pltpu.sync_copy(x_hbm.at[pl.ds(seq0, SEQ_PER_CHUNK)], idx_v)
    for s in range(SEQ_PER_CHUNK):
        for off, ln in SPLITS:
            pltpu.make_async_copy(
                tab_hbm.at[idx_v.at[s, pl.ds(off, ln)]],
                rows_v.at[pl.ds(s * MAXLEN + off, ln)],
                gsem,
            ).start()


def _wait_chunk(tab_hbm, idx_v, rows_v, gsem):
    for s in range(SEQ_PER_CHUNK):
        for off, ln in SPLITS:
            pltpu.make_async_copy(
                tab_hbm.at[idx_v.at[s, pl.ds(off, ln)]],
                rows_v.at[pl.ds(s * MAXLEN + off, ln)],
                gsem,
            ).wait()


def _add_pos(rows_v, pos_v):
    @pl.loop(0, MAXLEN)
    def pos_body(p):
        pv = [pos_v[p, pl.ds(j * LANES, LANES)] for j in range(VPE)]
        for s in range(SEQ_PER_CHUNK):
            r = s * MAXLEN + p
            for j in range(VPE):
                rows_v[r, pl.ds(j * LANES, LANES)] += pv[j]


def _store_chunk(rows_v, out_hbm, ssem, seq0, start):
    for s in range(SEQ_PER_CHUNK):
        cp = pltpu.make_async_copy(
            rows_v.at[pl.ds(s * MAXLEN, MAXLEN)],
            out_hbm.at[seq0 + s],
            ssem,
        )
        if start:
            cp.start()
        else:
            cp.wait()


def _body(x_hbm, tab_hbm, pos_hbm, out_hbm,
          idx0, idx1, idx2, idx3, rows0, rows1, rows2, rows3, pos_v,
          gsem0, gsem1, gsem2, gsem3, ssem0, ssem1, ssem2, ssem3):
    wid = lax.axis_index("s") * NC + lax.axis_index("c")
    base = wid * SEQ_PER_W
    idx = [idx0, idx1, idx2, idx3]
    rows = [rows0, rows1, rows2, rows3]
    gsem = [gsem0, gsem1, gsem2, gsem3]
    ssem = [ssem0, ssem1, ssem2, ssem3]

    pltpu.sync_copy(pos_hbm, pos_v)

    for b in range(LOOKAHEAD):
        _start_chunk(x_hbm, tab_hbm, idx[b], rows[b], gsem[b],
                     base + b * SEQ_PER_CHUNK)

    @pl.loop(0, N_CHUNK, step=NBUF)
    def chunk_body(c):
        for b in range(NBUF):
            seq0 = base + (c + b) * SEQ_PER_CHUNK
            _wait_chunk(tab_hbm, idx[b], rows[b], gsem[b])
            _add_pos(rows[b], pos_v)
            _store_chunk(rows[b], out_hbm, ssem[b], seq0, start=True)
            b2 = (b + LOOKAHEAD) % NBUF

            @pl.when(c + b + LOOKAHEAD < N_CHUNK)
            def _():
                @pl.when(c + b + LOOKAHEAD >= NBUF)
                def _():
                    _store_chunk(
                        rows[b2], out_hbm, ssem[b2],
                        base + (c + b + LOOKAHEAD - NBUF) * SEQ_PER_CHUNK,
                        start=False)
                _start_chunk(x_hbm, tab_hbm, idx[b2], rows[b2], gsem[b2],
                             base + (c + b + LOOKAHEAD) * SEQ_PER_CHUNK)

    for b in range(NBUF):
        cc = N_CHUNK - NBUF + b
        _store_chunk(rows[cc % NBUF], out_hbm, ssem[cc % NBUF],
                     base + cc * SEQ_PER_CHUNK, start=False)


@jax.jit
def _run(x, token_table, pos_table):
    mesh = plsc.VectorSubcoreMesh(core_axis_name="c", subcore_axis_name="s")
    f = functools.partial(
        pl.kernel,
        out_type=jax.ShapeDtypeStruct((BATCH, MAXLEN, EMBED), jnp.float32),
        mesh=mesh,
        scratch_types=(
            [pltpu.VMEM((SEQ_PER_CHUNK, MAXLEN), jnp.int32)] * NBUF
            + [pltpu.VMEM((CHUNK, EMBED), jnp.float32)] * NBUF
            + [pltpu.VMEM((MAXLEN, EMBED), jnp.float32)]
            + [pltpu.SemaphoreType.DMA] * (2 * NBUF)
        ),
        compiler_params=pltpu.CompilerParams(use_tc_tiling_on_sc=False),
    )(_body)
    return f(x, token_table, pos_table)


def kernel(x, token_table, pos_table):
    return _run(x.astype(jnp.int32), token_table, pos_table)
